# baseline (speedup 1.0000x reference)
import functools

import jax
import jax.numpy as jnp
from jax import lax
from jax.experimental import pallas as pl
from jax.experimental.pallas import tpu as pltpu
from jax.experimental.pallas import tpu_sc as plsc

NC = 2
NS = 16
NW = NC * NS

L = 16
IDXW = 128
GPC = 2
CHUNK = IDXW * GPC
NBUF = 2

TB = 128

_TC_TILED = pltpu.CompilerParams(
    use_tc_tiling_on_sc=True, needs_layout_passes=False
)


def _tpose_body(tt_hbm, tpad_hbm, in_v, out_v, sem_i, sem_o):
    d, v = tt_hbm.shape
    n_blocks = v // TB
    tail = v - n_blocks * TB
    wid = lax.axis_index("s") * NC + lax.axis_index("c")
    per_w = n_blocks // NW
    extra = n_blocks - per_w * NW
    my_n = per_w + jnp.where(wid < extra, 1, 0)
    blk0 = wid * per_w + jnp.minimum(wid, extra)

    iotas = [lax.broadcasted_iota(jnp.int32, (L,), 0) + L * g
             for g in range(TB // L)]

    def t_of(blk):
        return pl.multiple_of(blk * TB, TB)

    def start_in(blk, b):
        pltpu.async_copy(
            tt_hbm.at[:, pl.ds(t_of(blk), TB)], in_v.at[b], sem_i[b]
        )

    def wait_in(b):
        pltpu.make_async_copy(
            tt_hbm.at[:, pl.ds(0, TB)], in_v.at[b], sem_i[b]
        ).wait()

    def compute(b, ntok=TB):
        for c in range(d):
            cols = jnp.full((L,), c, jnp.int32)
            for g in range(ntok // L):
                x = in_v.at[b][c, pl.ds(L * g, L)]
                plsc.store_scatter(out_v.at[b], [iotas[g], cols], x)

    def start_out(blk, b):
        pltpu.async_copy(
            out_v.at[b], tpad_hbm.at[pl.ds(t_of(blk), TB)], sem_o[b]
        )

    def wait_out(b):
        pltpu.make_async_copy(
            out_v.at[b], tpad_hbm.at[pl.ds(0, TB)], sem_o[b]
        ).wait()

    for i in range(NBUF):
        start_in(blk0 + i, i)

    def body(j, _):
        blk = blk0 + j
        for bb in range(NBUF):
            @pl.when(jnp.mod(j, NBUF) == bb)
            def _():
                wait_in(bb)

                @pl.when(j >= NBUF)
                def _():
                    wait_out(bb)

                compute(bb)
                start_out(blk, bb)

                @pl.when(j + NBUF < my_n)
                def _():
                    start_in(blk + NBUF, bb)
        return 0

    lax.fori_loop(0, my_n, body, 0)
    for bb in range(NBUF):
        @pl.when(my_n > bb)
        def _():
            wait_out(bb)


def _gather_body(ids_hbm, table_hbm, out_hbm, idx_v, rows_v, sem_g, sem_o):
    b_total = out_hbm.shape[0]
    b_per_w = b_total // NW
    n_chunks = b_per_w // CHUNK
    n_grp = n_chunks // NBUF
    wid = lax.axis_index("s") * NC + lax.axis_index("c")
    row0 = wid * (b_per_w // IDXW)
    base0 = wid * b_per_w

    def start_gather(j, b):
        pltpu.sync_copy(ids_hbm.at[pl.ds(row0 + j * GPC, GPC)], idx_v.at[b])
        for r in range(GPC):
            pltpu.async_copy(
                table_hbm.at[idx_v.at[b].at[r]],
                rows_v.at[b].at[pl.ds(r * IDXW, IDXW)],
                sem_g[b],
            )

    def wait_gather(b):
        for r in range(GPC):
            pltpu.make_async_copy(
                table_hbm.at[pl.ds(0, IDXW)],
                rows_v.at[b].at[pl.ds(r * IDXW, IDXW)],
                sem_g[b],
            ).wait()

    def start_wb(j, b):
        pltpu.async_copy(
            rows_v.at[b], out_hbm.at[pl.ds(base0 + j * CHUNK, CHUNK)], sem_o[b]
        )

    def wait_wb(b):
        pltpu.make_async_copy(
            rows_v.at[b], out_hbm.at[pl.ds(0, CHUNK)], sem_o[b]
        ).wait()

    for b in range(NBUF):
        start_gather(b, b)

    def grp(g, _):
        for b in range(NBUF):
            j = g * NBUF + b
            wait_gather(b)
            start_wb(j, b)
            wait_wb(b)
            start_gather(j + NBUF, b)
        return 0

    lax.fori_loop(0, n_grp - 1, grp, 0)

    for b in range(NBUF):
        j = (n_grp - 1) * NBUF + b
        wait_gather(b)
        start_wb(j, b)
        wait_wb(b)


@functools.partial(jax.jit, static_argnames=())
def kernel(input_ids, table):
    batch, seq_len = input_ids.shape
    v, d = table.shape
    b = batch * seq_len
    dp = 128

    mesh = plsc.VectorSubcoreMesh(core_axis_name="c", subcore_axis_name="s")

    tpad = pl.kernel(
        _tpose_body,
        out_type=jax.ShapeDtypeStruct((v, dp), jnp.float32),
        mesh=mesh,
        scratch_types=[
            pltpu.VMEM((NBUF, d, TB), jnp.float32),
            pltpu.VMEM((NBUF, TB, dp), jnp.float32),
            [pltpu.SemaphoreType.DMA] * NBUF,
            [pltpu.SemaphoreType.DMA] * NBUF,
        ],
        compiler_params=_TC_TILED,
    )(table.T)
    v_main = (v // TB) * TB
    if v_main < v:
        tail_rows = jnp.pad(table[v_main:, :], ((0, 0), (0, dp - d)))
        tpad = lax.dynamic_update_slice(tpad, tail_rows, (v_main, 0))

    ids2d = input_ids.reshape(b // IDXW, IDXW)
    out = pl.kernel(
        _gather_body,
        out_type=jax.ShapeDtypeStruct((b, dp), jnp.float32),
        mesh=mesh,
        scratch_types=[
            pltpu.VMEM((NBUF, GPC, IDXW), jnp.int32),
            pltpu.VMEM((NBUF, CHUNK, dp), jnp.float32),
            [pltpu.SemaphoreType.DMA] * NBUF,
            [pltpu.SemaphoreType.DMA] * NBUF,
        ],
        compiler_params=_TC_TILED,
    )(ids2d, tpad)
    return out[:, :d].reshape(batch, seq_len, d)


# speedup vs baseline: 1.1304x; 1.1304x over previous
import functools

import jax
import jax.numpy as jnp
from jax import lax
from jax.experimental import pallas as pl
from jax.experimental.pallas import tpu as pltpu
from jax.experimental.pallas import tpu_sc as plsc

NC = 2
NS = 16
NW = NC * NS

L = 16
IDXW = 128
GPC = 2
CHUNK = IDXW * GPC
NBUF = 2

TB = 128

_TC_TILED = pltpu.CompilerParams(
    use_tc_tiling_on_sc=True, needs_layout_passes=False
)


def _tpose_body(tt_hbm, tpad_hbm, in_v, out_v, sem_i, sem_o):
    d, v = tt_hbm.shape
    dp = 128
    n_blocks = v // TB
    wid = lax.axis_index("s") * NC + lax.axis_index("c")
    per_w = n_blocks // NW
    extra = n_blocks - per_w * NW
    blk0 = wid * per_w
    n_grp = per_w // NBUF

    iotas = [(lax.broadcasted_iota(jnp.int32, (L,), 0) + L * g) * dp
             for g in range(TB // L)]

    def t_of(blk):
        return pl.multiple_of(blk * TB, TB)

    def start_in(blk, b):
        pltpu.async_copy(
            tt_hbm.at[:, pl.ds(t_of(blk), TB)], in_v[b], sem_i[b]
        )

    def wait_in(b):
        pltpu.make_async_copy(
            tt_hbm.at[:, pl.ds(0, TB)], in_v[b], sem_i[b]
        ).wait()

    def compute(b):
        def cbody(c, _):
            for g in range(TB // L):
                x = in_v[b][c, pl.ds(L * g, L)]
                plsc.store_scatter(out_v[b], [iotas[g] + c], x)
            return 0

        lax.fori_loop(0, d, cbody, 0)

    def start_out(blk, b):
        pltpu.async_copy(
            out_v[b], tpad_hbm.at[pl.ds(t_of(blk) * dp, TB * dp)], sem_o[b]
        )

    def wait_out(b):
        pltpu.make_async_copy(
            out_v[b], tpad_hbm.at[pl.ds(0, TB * dp)], sem_o[b]
        ).wait()

    for bb in range(NBUF):
        start_in(blk0 + bb, bb)
    for bb in range(NBUF):
        wait_in(bb)
        compute(bb)
        start_out(blk0 + bb, bb)
        start_in(blk0 + bb + NBUF, bb)

    def grp(g, _):
        for bb in range(NBUF):
            j = g * NBUF + bb
            wait_in(bb)
            wait_out(bb)
            compute(bb)
            start_out(blk0 + j, bb)
            start_in(blk0 + j + NBUF, bb)
        return 0

    lax.fori_loop(1, n_grp - 1, grp, 0)

    for bb in range(NBUF):
        j = (n_grp - 1) * NBUF + bb
        wait_in(bb)
        wait_out(bb)
        compute(bb)
        start_out(blk0 + j, bb)
    for bb in range(NBUF):
        wait_out(bb)

    @pl.when(wid < extra)
    def _():
        blk = n_blocks - extra + wid
        pltpu.sync_copy(tt_hbm.at[:, pl.ds(t_of(blk), TB)], in_v[0])
        compute(0)
        pltpu.sync_copy(out_v[0],
                        tpad_hbm.at[pl.ds(t_of(blk) * dp, TB * dp)])


def _gather_body(ids_hbm, table_hbm, out_hbm, idx_v, rows_v, sem_g, sem_o):
    b_total = out_hbm.shape[0]
    b_per_w = b_total // NW
    n_chunks = b_per_w // CHUNK
    n_grp = n_chunks // NBUF
    wid = lax.axis_index("s") * NC + lax.axis_index("c")
    row0 = wid * (b_per_w // IDXW)
    base0 = wid * b_per_w

    def start_gather(j, b):
        pltpu.sync_copy(ids_hbm.at[pl.ds(row0 + j * GPC, GPC)], idx_v.at[b])
        for r in range(GPC):
            pltpu.async_copy(
                table_hbm.at[idx_v.at[b].at[r]],
                rows_v.at[b].at[pl.ds(r * IDXW, IDXW)],
                sem_g[b],
            )

    def wait_gather(b):
        for r in range(GPC):
            pltpu.make_async_copy(
                table_hbm.at[pl.ds(0, IDXW)],
                rows_v.at[b].at[pl.ds(r * IDXW, IDXW)],
                sem_g[b],
            ).wait()

    def start_wb(j, b):
        pltpu.async_copy(
            rows_v.at[b], out_hbm.at[pl.ds(base0 + j * CHUNK, CHUNK)], sem_o[b]
        )

    def wait_wb(b):
        pltpu.make_async_copy(
            rows_v.at[b], out_hbm.at[pl.ds(0, CHUNK)], sem_o[b]
        ).wait()

    for b in range(NBUF):
        start_gather(b, b)

    def grp(g, _):
        for b in range(NBUF):
            j = g * NBUF + b
            wait_gather(b)
            start_wb(j, b)
            wait_wb(b)
            start_gather(j + NBUF, b)
        return 0

    lax.fori_loop(0, n_grp - 1, grp, 0)

    for b in range(NBUF):
        j = (n_grp - 1) * NBUF + b
        wait_gather(b)
        start_wb(j, b)
        wait_wb(b)


@functools.partial(jax.jit, static_argnames=())
def kernel(input_ids, table):
    batch, seq_len = input_ids.shape
    v, d = table.shape
    b = batch * seq_len
    dp = 128

    mesh = plsc.VectorSubcoreMesh(core_axis_name="c", subcore_axis_name="s")

    tpad = pl.kernel(
        _tpose_body,
        out_type=jax.ShapeDtypeStruct((v * dp,), jnp.float32),
        mesh=mesh,
        scratch_types=[
            [pltpu.VMEM((d, TB), jnp.float32)] * NBUF,
            [pltpu.VMEM((TB * dp,), jnp.float32)] * NBUF,
            [pltpu.SemaphoreType.DMA] * NBUF,
            [pltpu.SemaphoreType.DMA] * NBUF,
        ],
        compiler_params=_TC_TILED,
    )(table.T).reshape(v, dp)
    v_main = (v // TB) * TB
    if v_main < v:
        tail_rows = jnp.pad(table[v_main:, :], ((0, 0), (0, dp - d)))
        tpad = lax.dynamic_update_slice(tpad, tail_rows, (v_main, 0))

    ids2d = input_ids.reshape(b // IDXW, IDXW)
    out = pl.kernel(
        _gather_body,
        out_type=jax.ShapeDtypeStruct((b, dp), jnp.float32),
        mesh=mesh,
        scratch_types=[
            pltpu.VMEM((NBUF, GPC, IDXW), jnp.int32),
            pltpu.VMEM((NBUF, CHUNK, dp), jnp.float32),
            [pltpu.SemaphoreType.DMA] * NBUF,
            [pltpu.SemaphoreType.DMA] * NBUF,
        ],
        compiler_params=_TC_TILED,
    )(ids2d, tpad)
    return out[:, :d].reshape(batch, seq_len, d)


# speedup vs baseline: 1.9577x; 1.7319x over previous
import functools

import jax
import jax.numpy as jnp
from jax import lax
from jax.experimental import pallas as pl
from jax.experimental.pallas import tpu as pltpu
from jax.experimental.pallas import tpu_sc as plsc

NC = 2
NS = 16
NW = NC * NS

IDXW = 128
GPC = 2
CHUNK = IDXW * GPC
NBUF = 2


def _emb_body(ids_hbm, table_hbm, out_hbm, idx_v, rows_v, sem_g, sem_o):
    dp = table_hbm.shape[1]
    d = 64
    b_total = out_hbm.shape[0]
    b_per_w = b_total // NW
    n_chunks = b_per_w // CHUNK
    n_grp = n_chunks // NBUF
    wid = lax.axis_index("s") * NC + lax.axis_index("c")
    row0 = wid * (b_per_w // IDXW)
    base0 = wid * b_per_w

    def start_gather(j, b):
        pltpu.sync_copy(ids_hbm.at[pl.ds(row0 + j * GPC, GPC)], idx_v.at[b])
        for r in range(GPC):
            pltpu.async_copy(
                table_hbm.at[idx_v.at[b].at[r]],
                rows_v.at[b].at[pl.ds(r * IDXW, IDXW)],
                sem_g[b],
            )

    def wait_gather(b):
        for r in range(GPC):
            pltpu.make_async_copy(
                table_hbm.at[pl.ds(0, IDXW)],
                rows_v.at[b].at[pl.ds(r * IDXW, IDXW)],
                sem_g[b],
            ).wait()

    def start_wb(j, b):
        pltpu.async_copy(
            rows_v.at[b].at[pl.ds(0, CHUNK), pl.ds(0, d)],
            out_hbm.at[pl.ds(base0 + j * CHUNK, CHUNK), pl.ds(0, d)],
            sem_o[b],
        )

    def wait_wb(b):
        pltpu.make_async_copy(
            rows_v.at[b].at[pl.ds(0, CHUNK), pl.ds(0, d)],
            out_hbm.at[pl.ds(0, CHUNK), pl.ds(0, d)],
            sem_o[b],
        ).wait()

    for b in range(NBUF):
        start_gather(b, b)

    def grp(g, _):
        for b in range(NBUF):
            j = g * NBUF + b
            wait_gather(b)
            start_wb(j, b)
            wait_wb(b)
            start_gather(j + NBUF, b)
        return 0

    lax.fori_loop(0, n_grp - 1, grp, 0)

    for b in range(NBUF):
        j = (n_grp - 1) * NBUF + b
        wait_gather(b)
        start_wb(j, b)
        wait_wb(b)


@functools.partial(jax.jit, static_argnames=())
def kernel(input_ids, table):
    batch, seq_len = input_ids.shape
    v, d = table.shape
    b = batch * seq_len
    dp = 128

    ids2d = input_ids.reshape(b // IDXW, IDXW)
    tpad = jnp.pad(table, ((0, 0), (0, dp - d)))

    mesh = plsc.VectorSubcoreMesh(core_axis_name="c", subcore_axis_name="s")
    out = pl.kernel(
        _emb_body,
        out_type=jax.ShapeDtypeStruct((b, dp), jnp.float32),
        mesh=mesh,
        scratch_types=[
            pltpu.VMEM((NBUF, GPC, IDXW), jnp.int32),
            pltpu.VMEM((NBUF, CHUNK, dp), jnp.float32),
            [pltpu.SemaphoreType.DMA] * NBUF,
            [pltpu.SemaphoreType.DMA] * NBUF,
        ],
        compiler_params=pltpu.CompilerParams(use_tc_tiling_on_sc=False),
    )(ids2d, tpad)
    return out[:, :d].reshape(batch, seq_len, d)


# speedup vs baseline: 2.5575x; 1.3063x over previous
import functools

import jax
import jax.numpy as jnp
from jax import lax
from jax.experimental import pallas as pl
from jax.experimental.pallas import tpu as pltpu
from jax.experimental.pallas import tpu_sc as plsc

NC = 2
NS = 16
NW = NC * NS

L = 16
IDXW = 128
GPC = 3
CHUNK = IDXW * GPC
NBUF = 2

TB = 128
DR = 72

_TC_TILED = pltpu.CompilerParams(
    use_tc_tiling_on_sc=True, needs_layout_passes=False
)


def _tpose_body(tt_hbm, tpad_hbm, in_v, out_v, sem_i, sem_o):
    d, v = tt_hbm.shape
    n_blocks = v // TB
    wid = lax.axis_index("s") * NC + lax.axis_index("c")
    per_w = n_blocks // NW
    extra = n_blocks - per_w * NW
    blk0 = wid * per_w
    n_grp = per_w // NBUF

    iotas = [(lax.broadcasted_iota(jnp.int32, (L,), 0) + L * g) * DR
             for g in range(TB // L)]

    def t_of(blk):
        return pl.multiple_of(blk * TB, TB)

    def start_in(blk, b):
        pltpu.async_copy(
            tt_hbm.at[:, pl.ds(t_of(blk), TB)], in_v[b], sem_i[b]
        )

    def wait_in(b):
        pltpu.make_async_copy(
            tt_hbm.at[:, pl.ds(0, TB)], in_v[b], sem_i[b]
        ).wait()

    def compute(b):
        def cbody(c, _):
            xs = [in_v[b][c, pl.ds(L * g, L)] for g in range(TB // L)]
            idxs = [iotas[g] + c for g in range(TB // L)]
            for g in range(TB // L):
                plsc.store_scatter(out_v[b], [idxs[g]], xs[g])
            return 0

        lax.fori_loop(0, d, cbody, 0)

    def start_out(blk, b):
        pltpu.async_copy(
            out_v[b], tpad_hbm.at[pl.ds(t_of(blk) * DR, TB * DR)], sem_o[b]
        )

    def wait_out(b):
        pltpu.make_async_copy(
            out_v[b], tpad_hbm.at[pl.ds(0, TB * DR)], sem_o[b]
        ).wait()

    for bb in range(NBUF):
        start_in(blk0 + bb, bb)
    for bb in range(NBUF):
        wait_in(bb)
        compute(bb)
        start_out(blk0 + bb, bb)
        start_in(blk0 + bb + NBUF, bb)

    def grp(g, _):
        for bb in range(NBUF):
            j = g * NBUF + bb
            wait_in(bb)
            wait_out(bb)
            compute(bb)
            start_out(blk0 + j, bb)
            start_in(blk0 + j + NBUF, bb)
        return 0

    lax.fori_loop(1, n_grp - 1, grp, 0)

    for bb in range(NBUF):
        j = (n_grp - 1) * NBUF + bb
        wait_in(bb)
        wait_out(bb)
        compute(bb)
        start_out(blk0 + j, bb)
    for bb in range(NBUF):
        wait_out(bb)

    @pl.when(wid < extra)
    def _():
        blk = n_blocks - extra + wid
        pltpu.sync_copy(tt_hbm.at[:, pl.ds(t_of(blk), TB)], in_v[0])
        compute(0)
        pltpu.sync_copy(out_v[0],
                        tpad_hbm.at[pl.ds(t_of(blk) * DR, TB * DR)])


def _gather_body(ids_hbm, table_hbm, out_hbm, idx_v, rows_v, sem_g, sem_o):
    d = 64
    b_total = out_hbm.shape[0]
    b_per_w = b_total // NW
    n_chunks = b_per_w // CHUNK
    n_grp = n_chunks // NBUF
    wid = lax.axis_index("s") * NC + lax.axis_index("c")
    row0 = wid * (b_per_w // IDXW)
    base0 = wid * b_per_w

    def start_gather(j, b):
        pltpu.sync_copy(ids_hbm.at[pl.ds(row0 + j * GPC, GPC)], idx_v.at[b])
        for r in range(GPC):
            pltpu.async_copy(
                table_hbm.at[idx_v.at[b].at[r]],
                rows_v.at[b].at[pl.ds(r * IDXW, IDXW)],
                sem_g[b],
            )

    def wait_gather(b):
        for r in range(GPC):
            pltpu.make_async_copy(
                table_hbm.at[pl.ds(0, IDXW)],
                rows_v.at[b].at[pl.ds(r * IDXW, IDXW)],
                sem_g[b],
            ).wait()

    def start_wb(j, b):
        pltpu.async_copy(
            rows_v.at[b],
            out_hbm.at[pl.ds(base0 + j * CHUNK, CHUNK), pl.ds(0, DR)],
            sem_o[b],
        )

    def wait_wb(b):
        pltpu.make_async_copy(
            rows_v.at[b], out_hbm.at[pl.ds(0, CHUNK), pl.ds(0, DR)], sem_o[b]
        ).wait()

    for b in range(NBUF):
        start_gather(b, b)

    def grp(g, _):
        for b in range(NBUF):
            j = g * NBUF + b
            wait_gather(b)
            start_wb(j, b)
            wait_wb(b)
            start_gather(j + NBUF, b)
        return 0

    lax.fori_loop(0, n_grp - 1, grp, 0)

    for b in range(NBUF):
        j = (n_grp - 1) * NBUF + b
        wait_gather(b)
        start_wb(j, b)
        wait_wb(b)


@functools.partial(jax.jit, static_argnames=())
def kernel(input_ids, table):
    batch, seq_len = input_ids.shape
    v, d = table.shape
    b = batch * seq_len
    dp = 128

    mesh = plsc.VectorSubcoreMesh(core_axis_name="c", subcore_axis_name="s")

    tpad = pl.kernel(
        _tpose_body,
        out_type=jax.ShapeDtypeStruct((v * DR,), jnp.float32),
        mesh=mesh,
        scratch_types=[
            [pltpu.VMEM((d, TB), jnp.float32)] * NBUF,
            [pltpu.VMEM((TB * DR,), jnp.float32)] * NBUF,
            [pltpu.SemaphoreType.DMA] * NBUF,
            [pltpu.SemaphoreType.DMA] * NBUF,
        ],
        compiler_params=_TC_TILED,
    )(table.T)
    v_main = (v // TB) * TB
    if v_main < v:
        tail = jnp.pad(table[v_main:, :], ((0, 0), (0, DR - d))).reshape(-1)
        tpad = lax.dynamic_update_slice(tpad, tail, (v_main * DR,))
    tpad = tpad.reshape(v, DR)

    ids2d = input_ids.reshape(b // IDXW, IDXW)
    out = pl.kernel(
        _gather_body,
        out_type=jax.ShapeDtypeStruct((b, dp), jnp.float32),
        mesh=mesh,
        scratch_types=[
            pltpu.VMEM((NBUF, GPC, IDXW), jnp.int32),
            pltpu.VMEM((NBUF, CHUNK, DR), jnp.float32),
            [pltpu.SemaphoreType.DMA] * NBUF,
            [pltpu.SemaphoreType.DMA] * NBUF,
        ],
        compiler_params=pltpu.CompilerParams(use_tc_tiling_on_sc=False),
    )(ids2d, tpad)
    return out[:, :d].reshape(batch, seq_len, d)


# speedup vs baseline: 2.5854x; 1.0109x over previous
import functools

import jax
import jax.numpy as jnp
from jax import lax
from jax.experimental import pallas as pl
from jax.experimental.pallas import tpu as pltpu
from jax.experimental.pallas import tpu_sc as plsc

NC = 2
NS = 16
NW = NC * NS

L = 16
IDXW = 128
GPC = 4
CHUNK = IDXW * GPC
NBUF = 2

TB = 128
DR = 72

_TC_TILED = pltpu.CompilerParams(
    use_tc_tiling_on_sc=True, needs_layout_passes=False
)


def _tpose_body(tt_hbm, tpad_hbm, in_v, out_v, sem_i, sem_o):
    d, v = tt_hbm.shape
    n_blocks = v // TB
    wid = lax.axis_index("s") * NC + lax.axis_index("c")
    per_w = n_blocks // NW
    extra = n_blocks - per_w * NW
    blk0 = wid * per_w
    n_grp = per_w // NBUF

    iotas = [(lax.broadcasted_iota(jnp.int32, (L,), 0) + L * g) * DR
             for g in range(TB // L)]

    def t_of(blk):
        return pl.multiple_of(blk * TB, TB)

    def start_in(blk, b):
        pltpu.async_copy(
            tt_hbm.at[:, pl.ds(t_of(blk), TB)], in_v[b], sem_i[b]
        )

    def wait_in(b):
        pltpu.make_async_copy(
            tt_hbm.at[:, pl.ds(0, TB)], in_v[b], sem_i[b]
        ).wait()

    def compute(b):
        def cbody(c, _):
            xs = [in_v[b][c, pl.ds(L * g, L)] for g in range(TB // L)]
            idxs = [iotas[g] + c for g in range(TB // L)]
            for g in range(TB // L):
                plsc.store_scatter(out_v[b], [idxs[g]], xs[g])
            return 0

        lax.fori_loop(0, d, cbody, 0)

    def start_out(blk, b):
        pltpu.async_copy(
            out_v[b], tpad_hbm.at[pl.ds(t_of(blk) * DR, TB * DR)], sem_o[b]
        )

    def wait_out(b):
        pltpu.make_async_copy(
            out_v[b], tpad_hbm.at[pl.ds(0, TB * DR)], sem_o[b]
        ).wait()

    for bb in range(NBUF):
        start_in(blk0 + bb, bb)
    for bb in range(NBUF):
        wait_in(bb)
        compute(bb)
        start_out(blk0 + bb, bb)
        start_in(blk0 + bb + NBUF, bb)

    def grp(g, _):
        for bb in range(NBUF):
            j = g * NBUF + bb
            wait_in(bb)
            wait_out(bb)
            compute(bb)
            start_out(blk0 + j, bb)
            start_in(blk0 + j + NBUF, bb)
        return 0

    lax.fori_loop(1, n_grp - 1, grp, 0)

    for bb in range(NBUF):
        j = (n_grp - 1) * NBUF + bb
        wait_in(bb)
        wait_out(bb)
        compute(bb)
        start_out(blk0 + j, bb)
    for bb in range(NBUF):
        wait_out(bb)

    @pl.when(wid < extra)
    def _():
        blk = n_blocks - extra + wid
        pltpu.sync_copy(tt_hbm.at[:, pl.ds(t_of(blk), TB)], in_v[0])
        compute(0)
        pltpu.sync_copy(out_v[0],
                        tpad_hbm.at[pl.ds(t_of(blk) * DR, TB * DR)])


def _gather_body(ids_hbm, table_hbm, out_hbm, idx_v, rows_v, sem_g, sem_o):
    d = 64
    b_total = out_hbm.shape[0]
    b_per_w = b_total // NW
    n_chunks = b_per_w // CHUNK
    n_grp = n_chunks // NBUF
    wid = lax.axis_index("s") * NC + lax.axis_index("c")
    row0 = wid * (b_per_w // IDXW)
    base0 = wid * b_per_w

    def start_gather(j, b):
        pltpu.sync_copy(ids_hbm.at[pl.ds(row0 + j * GPC, GPC)], idx_v.at[b])
        for r in range(GPC):
            pltpu.async_copy(
                table_hbm.at[idx_v.at[b].at[r]],
                rows_v.at[b].at[pl.ds(r * IDXW, IDXW)],
                sem_g[b],
            )

    def wait_gather(b):
        for r in range(GPC):
            pltpu.make_async_copy(
                table_hbm.at[pl.ds(0, IDXW)],
                rows_v.at[b].at[pl.ds(r * IDXW, IDXW)],
                sem_g[b],
            ).wait()

    def start_wb(j, b):
        pltpu.async_copy(
            rows_v.at[b],
            out_hbm.at[pl.ds(base0 + j * CHUNK, CHUNK), pl.ds(0, DR)],
            sem_o[b],
        )

    def wait_wb(b):
        pltpu.make_async_copy(
            rows_v.at[b], out_hbm.at[pl.ds(0, CHUNK), pl.ds(0, DR)], sem_o[b]
        ).wait()

    for b in range(NBUF):
        start_gather(b, b)

    def grp(g, _):
        for b in range(NBUF):
            j = g * NBUF + b
            wait_gather(b)
            start_wb(j, b)
            wait_wb(b)
            start_gather(j + NBUF, b)
        return 0

    lax.fori_loop(0, n_grp - 1, grp, 0)

    for b in range(NBUF):
        j = (n_grp - 1) * NBUF + b
        wait_gather(b)
        start_wb(j, b)
        wait_wb(b)


@functools.partial(jax.jit, static_argnames=())
def kernel(input_ids, table):
    batch, seq_len = input_ids.shape
    v, d = table.shape
    b = batch * seq_len
    dp = 128

    mesh = plsc.VectorSubcoreMesh(core_axis_name="c", subcore_axis_name="s")

    tpad = pl.kernel(
        _tpose_body,
        out_type=jax.ShapeDtypeStruct((v * DR,), jnp.float32),
        mesh=mesh,
        scratch_types=[
            [pltpu.VMEM((d, TB), jnp.float32)] * NBUF,
            [pltpu.VMEM((TB * DR,), jnp.float32)] * NBUF,
            [pltpu.SemaphoreType.DMA] * NBUF,
            [pltpu.SemaphoreType.DMA] * NBUF,
        ],
        compiler_params=_TC_TILED,
    )(table.T)
    v_main = (v // TB) * TB
    if v_main < v:
        tail = jnp.pad(table[v_main:, :], ((0, 0), (0, DR - d))).reshape(-1)
        tpad = lax.dynamic_update_slice(tpad, tail, (v_main * DR,))
    tpad = tpad.reshape(v, DR)

    ids2d = input_ids.reshape(b // IDXW, IDXW)
    out = pl.kernel(
        _gather_body,
        out_type=jax.ShapeDtypeStruct((b, dp), jnp.float32),
        mesh=mesh,
        scratch_types=[
            pltpu.VMEM((NBUF, GPC, IDXW), jnp.int32),
            pltpu.VMEM((NBUF, CHUNK, DR), jnp.float32),
            [pltpu.SemaphoreType.DMA] * NBUF,
            [pltpu.SemaphoreType.DMA] * NBUF,
        ],
        compiler_params=pltpu.CompilerParams(use_tc_tiling_on_sc=False),
    )(ids2d, tpad)
    return out[:, :d].reshape(batch, seq_len, d)
